# baseline (speedup 1.0000x reference)
import functools

import numpy as np
import jax
import jax.numpy as jnp
from jax import lax
from jax.experimental import pallas as pl
from jax.experimental.pallas import tpu as pltpu
from jax.experimental.pallas import tpu_sc as plsc

_BINS = 6
_WIDTH = np.float32(256.0 / _BINS)
_OUT = 240
_BIG = np.float32(3.0e38)
_NW = 32
_L = 16

_LAYERS = ((112, 112, 3, 8), (56, 56, 6, 7), (28, 28, 12, 7))

_LOG2C = (-0.02482598, 0.26686277, -1.2342799, 3.21886981,
          -5.26415552, 6.06585886, -3.02832497)
_LN2 = 0.6931471805599453


def _resize_matrix(src):
    x = (np.arange(_OUT, dtype=np.float64) + 0.5) * (src / _OUT) - 0.5
    lo = np.floor(x).astype(np.int64)
    frac = x - lo
    a = np.zeros((_OUT, src), np.float64)
    for i in range(_OUT):
        for tap, wt in ((lo[i], 1.0 - frac[i]), (lo[i] + 1, frac[i])):
            a[i, min(max(int(tap), 0), src - 1)] += wt
    return a.astype(np.float32)


def _fmin6(vals):
    return functools.reduce(jnp.minimum, vals)


def _fmax6(vals):
    return functools.reduce(jnp.maximum, vals)


def _ln16(x):
    bits = lax.bitcast_convert_type(x, jnp.int32)
    e = ((bits >> 23) & 0xFF).astype(jnp.float32) - 127.0
    m = lax.bitcast_convert_type((bits & 0x007FFFFF) | 0x3F800000,
                                 jnp.float32)
    p = jnp.full((_L,), _LOG2C[0], jnp.float32)
    for c in _LOG2C[1:]:
        p = p * m + c
    return (e + p) * _LN2


def _lane_shuffle(v, sh):
    idx = lax.broadcasted_iota(jnp.int32, (_L,), 0) ^ sh
    return v.at[idx].get(mode="promise_in_bounds")


def _splat_min(v):
    for sh in (8, 4, 2, 1):
        v = jnp.minimum(v, _lane_shuffle(v, sh))
    return v


def _splat_max(v):
    for sh in (8, 4, 2, 1):
        v = jnp.maximum(v, _lane_shuffle(v, sh))
    return v


def _minmax_pass(buf, bm, nchunk, u):
    def body(i, carry):
        mn, mx = carry
        for t in range(u):
            ds = pl.ds((i * u + t) * _L, _L)
            v0 = buf[ds] * bm[ds]
            mn = jnp.minimum(mn, v0)
            mx = jnp.maximum(mx, v0)
        return mn, mx

    init = jnp.full((_L,), _BIG, jnp.float32)
    mn, mx = lax.fori_loop(0, nchunk // u, body, (init, -init))
    return _splat_min(mn), _splat_max(mx)


def _splat_sum(v):
    for sh in (8, 4, 2, 1):
        v = v + _lane_shuffle(v, sh)
    return v


def _hist_pass(buf, bm, hid, nchunk, u, tmin_s, scale_s):
    one = jnp.full((_L,), 1.0, jnp.float32)
    zro = jnp.zeros((_L,), jnp.float32)

    def body(i, carry):
        c1 = list(carry[:_BINS])
        c2 = list(carry[_BINS:])
        for t in range(u):
            ds = pl.ds((i * u + t) * _L, _L)
            m = bm[ds]
            ch = (buf[ds] * m - tmin_s) * scale_s
            b1 = jnp.minimum((ch * (1.0 / _WIDTH)).astype(jnp.int32), 5)
            h = jnp.clip((ch * 6.0 - 1.0).astype(jnp.int32), 0, 5)
            for b in range(_BINS):
                c1[b] = c1[b] + jnp.where(b1 == b, one, zro)
                c2[b] = c2[b] + jnp.where(h == b, one, zro)
            hid[ds] = jnp.where(m == 0.0, _BINS, h)
        return tuple(c1) + tuple(c2)

    out = lax.fori_loop(0, nchunk // u, body, (zro,) * (2 * _BINS))
    return ([_splat_sum(out[b]) for b in range(_BINS)],
            [_splat_sum(out[_BINS + b]) for b in range(_BINS)])


def _chan_table(c1i, c2i, tmin_s, scale_s, n, nb, is0_s):
    c1 = list(c1i)
    c2 = list(c2i)
    chb = (0.0 - tmin_s) * scale_s
    hb = jnp.clip((chb * 6.0 - 1.0).astype(jnp.int32), 0, 5)

    lv = [-_ln16(c1[b] * (1.0 / n) + 1e-4) for b in range(_BINS)]
    pres = [c2[b] > 0.0 for b in range(_BINS)]
    dmin = _fmin6([jnp.where(pres[b], lv[b], _BIG) for b in range(_BINS)])
    dmax = _fmax6([jnp.where(pres[b], lv[b], -_BIG) for b in range(_BINS)])
    drng = dmax - dmin
    ddeg = drng == 0.0
    dsafe = jnp.where(ddeg, 1.0, drng)
    ln = [jnp.where(ddeg, 0.0, (lv[b] - dmin) / dsafe) for b in range(_BINS)]
    lmax = _fmax6([jnp.where(pres[b], ln[b], -_BIG) for b in range(_BINS)])
    lmean = sum(c2[b] * ln[b] for b in range(_BINS)) * (1.0 / n)
    w_r = (lmax - lmean) * (lmax - lmean)
    rv = [ln[b] * w_r for b in range(_BINS)]

    rminp = _fmin6([jnp.where(pres[b], rv[b], _BIG) for b in range(_BINS)])
    rmaxp = _fmax6([jnp.where(pres[b], rv[b], -_BIG) for b in range(_BINS)])
    rmean = sum(c2[b] * rv[b] for b in range(_BINS)) * (1.0 / n)
    w0 = (rmaxp - rmean) * (rmaxp - rmean)
    frng = rmaxp - rminp
    fdeg = frng == 0.0
    fsafe = jnp.where(fdeg, 1.0, frng)
    t0 = [jnp.where(fdeg, 0.0, (rv[b] - rminp) / fsafe * w0)
          for b in range(_BINS)]
    sb0 = sum(jnp.where(hb == b, t0[b], 0.0) for b in range(_BINS))

    cint = [c2[b] - nb * jnp.where(hb == b, 1.0, 0.0) for b in range(_BINS)]
    presi = [cint[b] > 0.0 for b in range(_BINS)]
    zmin = jnp.minimum(
        0.0, _fmin6([jnp.where(presi[b], rv[b], _BIG) for b in range(_BINS)]))
    zmax = jnp.maximum(
        0.0, _fmax6([jnp.where(presi[b], rv[b], -_BIG) for b in range(_BINS)]))
    zmean = sum(cint[b] * rv[b] for b in range(_BINS)) * (1.0 / n)
    wz = (zmax - zmean) * (zmax - zmean)
    zrng = zmax - zmin
    zdeg = zrng == 0.0
    zsafe = jnp.where(zdeg, 1.0, zrng)
    tz = [jnp.where(zdeg, 0.0, (rv[b] - zmin) / zsafe * wz)
          for b in range(_BINS)]
    bz = jnp.where(zdeg, 0.0, (0.0 - zmin) / zsafe * wz)

    tab = [tz[b] + (t0[b] - tz[b]) * is0_s for b in range(_BINS)]
    sb = bz + (sb0 - bz) * is0_s
    return tab, sb


def _apply_pass(hid, acc, nchunk, u, tab, sb):
    def body(i, _):
        for t in range(u):
            ds = pl.ds((i * u + t) * _L, _L)
            h = hid[ds]
            g = sb
            for b in range(_BINS):
                g = jnp.where(h == b, tab[b], g)
            acc[ds] = acc[ds] + g
        return 0

    lax.fori_loop(0, nchunk // u, body, 0)


def _sc_layer(hbm, bmh, out, buf, bm, hid, acc, wid, hdim, wdim, per, u):
    hw = hdim * wdim
    nchunk = hw // _L
    nf = float(hw)
    nbf = float(2 * hdim + 2 * wdim - 4)

    def zero_body(i, _):
        for t in range(u):
            ds = pl.ds((i * u + t) * _L, _L)
            acc[ds] = jnp.zeros((_L,), jnp.float32)
        return 0

    lax.fori_loop(0, nchunk // u, zero_body, 0)
    pltpu.sync_copy(bmh, bm)

    def chan_body(k, _):
        c = wid * per + k
        pltpu.sync_copy(hbm.at[pl.ds(c * hw, hw)], buf)
        tmin_s, tmax_s = _minmax_pass(buf, bm, nchunk, u)
        rng = tmax_s - tmin_s
        deg = rng == 0.0
        scale_s = jnp.where(deg, 0.0, 256.0 / jnp.where(deg, 1.0, rng))
        c1, c2 = _hist_pass(buf, bm, hid, nchunk, u, tmin_s, scale_s)
        is0_s = jnp.where(c == 0, 1.0, 0.0).astype(jnp.float32)
        tab, sb = _chan_table(c1, c2, tmin_s, scale_s, nf, nbf, is0_s)
        _apply_pass(hid, acc, nchunk, u, tab, sb)
        return 0

    lax.fori_loop(0, per, chan_body, 0)
    pltpu.sync_copy(acc, out.at[pl.ds(wid * hw, hw)])


def _make_sc_call():
    mesh = plsc.VectorSubcoreMesh(core_axis_name="c", subcore_axis_name="s")
    out_type = [jax.ShapeDtypeStruct((_NW * h * w,), jnp.float32)
                for h, w, _, _ in _LAYERS]
    scratch = []
    for h, w, _, _ in _LAYERS:
        hw = h * w
        scratch += [pltpu.VMEM((hw,), jnp.float32),
                    pltpu.VMEM((hw,), jnp.float32),
                    pltpu.VMEM((hw,), jnp.int32),
                    pltpu.VMEM((hw,), jnp.float32)]

    @functools.partial(pl.kernel, mesh=mesh, out_type=out_type,
                       scratch_types=scratch)
    def body(l0, l1, l2, mb0, mb1, mb2, p0, p1, p2,
             buf0, bm0, hid0, acc0,
             buf1, bm1, hid1, acc1,
             buf2, bm2, hid2, acc2):
        wid = lax.axis_index("s") * 2 + lax.axis_index("c")
        hbms = (l0, l1, l2)
        masks = (mb0, mb1, mb2)
        outs = (p0, p1, p2)
        scr = ((buf0, bm0, hid0, acc0),
               (buf1, bm1, hid1, acc1),
               (buf2, bm2, hid2, acc2))
        for li, (hdim, wdim, per, u) in enumerate(_LAYERS):
            buf, bmm, hid, acc = scr[li]
            _sc_layer(hbms[li], masks[li], outs[li], buf, bmm, hid, acc,
                      wid, hdim, wdim, per, u)

    return body


_SC_CALL = _make_sc_call()


def _tail_body(p_ref, a_ref, at_ref, col_ref):
    p = jnp.sum(p_ref[...], axis=0)
    pmin = jnp.min(p)
    pmax = jnp.max(p)
    prng = pmax - pmin
    pdeg = prng == 0.0
    psafe = jnp.where(pdeg, 1.0, prng)
    pn = jnp.where(pdeg, 0.0, (p - pmin) / psafe)
    pt = jnp.where(pn < 0.2, 0.0, pn)
    tmp = jnp.dot(a_ref[...], pt, preferred_element_type=jnp.float32)
    r = jnp.dot(tmp, at_ref[...], preferred_element_type=jnp.float32)
    rmin = jnp.min(r)
    rmax = jnp.max(r)
    rrng = rmax - rmin
    rdeg = rrng == 0.0
    rsafe = jnp.where(rdeg, 1.0, rrng)
    col_ref[...] = jnp.where(rdeg, 0.0, (r - rmin) / rsafe * 256.0)


def _tail_col(p, amat, atmat):
    nw, h, w = p.shape
    return pl.pallas_call(
        _tail_body,
        grid=(1,),
        in_specs=[
            pl.BlockSpec((nw, h, w), lambda i: (0, 0, 0)),
            pl.BlockSpec((_OUT, h), lambda i: (0, 0)),
            pl.BlockSpec((h, _OUT), lambda i: (0, 0)),
        ],
        out_specs=pl.BlockSpec((_OUT, _OUT), lambda i: (0, 0)),
        out_shape=jax.ShapeDtypeStruct((_OUT, _OUT), jnp.float32),
    )(p, amat, atmat)


_A112 = _resize_matrix(112)
_A56 = _resize_matrix(56)
_A28 = _resize_matrix(28)


def _border_mask(h, w):
    m = np.ones((h, w), np.float32)
    m[0, :] = 0.0
    m[-1, :] = 0.0
    m[:, 0] = 0.0
    m[:, -1] = 0.0
    return m.reshape(-1)


_BMASKS = tuple(_border_mask(h, w) for h, w, _, _ in _LAYERS)


def kernel(layer0, layer1, layer2):
    flats = [x[0].reshape(x.shape[1] * x.shape[2] * x.shape[3])
             for x in (layer0, layer1, layer2)]
    parts = _SC_CALL(*flats, *(jnp.asarray(m) for m in _BMASKS))
    cols = []
    for part, (h, w, _, _), a in zip(parts, _LAYERS, (_A112, _A56, _A28)):
        cols.append(_tail_col(
            part.reshape(_NW, h, w),
            jnp.asarray(a), jnp.asarray(np.ascontiguousarray(a.T))))
    groups = jnp.stack(cols, axis=-1)
    return groups.sum(axis=-1), groups


# speedup vs baseline: 1.1975x; 1.1975x over previous
import functools

import numpy as np
import jax
import jax.numpy as jnp
from jax import lax
from jax.experimental import pallas as pl
from jax.experimental.pallas import tpu as pltpu
from jax.experimental.pallas import tpu_sc as plsc

_BINS = 6
_WIDTH = np.float32(256.0 / _BINS)
_OUT = 240
_BIG = np.float32(3.0e38)
_NW = 32
_L = 16

_LAYERS = ((112, 112, 3, 8), (56, 56, 6, 7), (28, 28, 12, 7))

_LOG2C = (-0.02482598, 0.26686277, -1.2342799, 3.21886981,
          -5.26415552, 6.06585886, -3.02832497)
_LN2 = 0.6931471805599453


def _resize_matrix(src):
    x = (np.arange(_OUT, dtype=np.float64) + 0.5) * (src / _OUT) - 0.5
    lo = np.floor(x).astype(np.int64)
    frac = x - lo
    a = np.zeros((_OUT, src), np.float64)
    for i in range(_OUT):
        for tap, wt in ((lo[i], 1.0 - frac[i]), (lo[i] + 1, frac[i])):
            a[i, min(max(int(tap), 0), src - 1)] += wt
    return a.astype(np.float32)


def _fmin6(vals):
    return functools.reduce(jnp.minimum, vals)


def _fmax6(vals):
    return functools.reduce(jnp.maximum, vals)


def _ln16(x):
    bits = lax.bitcast_convert_type(x, jnp.int32)
    e = ((bits >> 23) & 0xFF).astype(jnp.float32) - 127.0
    m = lax.bitcast_convert_type((bits & 0x007FFFFF) | 0x3F800000,
                                 jnp.float32)
    p = jnp.full((_L,), _LOG2C[0], jnp.float32)
    for c in _LOG2C[1:]:
        p = p * m + c
    return (e + p) * _LN2


def _lane_shuffle(v, sh):
    idx = lax.broadcasted_iota(jnp.int32, (_L,), 0) ^ sh
    return v.at[idx].get(mode="promise_in_bounds")


def _splat_min(v):
    for sh in (8, 4, 2, 1):
        v = jnp.minimum(v, _lane_shuffle(v, sh))
    return v


def _splat_max(v):
    for sh in (8, 4, 2, 1):
        v = jnp.maximum(v, _lane_shuffle(v, sh))
    return v


def _minmax_pass(buf, bm, nchunk, u):
    def body(i, carry):
        mn, mx = carry
        for t in range(u):
            ds = pl.ds((i * u + t) * _L, _L)
            v0 = buf[ds] * bm[ds]
            mn = jnp.minimum(mn, v0)
            mx = jnp.maximum(mx, v0)
        return mn, mx

    init = jnp.full((_L,), _BIG, jnp.float32)
    mn, mx = lax.fori_loop(0, nchunk // u, body, (init, -init))
    return _splat_min(mn), _splat_max(mx)


def _splat_sum(v):
    for sh in (8, 4, 2, 1):
        v = v + _lane_shuffle(v, sh)
    return v


def _hist_pass(buf, bm, hid, nchunk, u, tmin_s, scale_s, nf):
    one = jnp.full((_L,), 1.0, jnp.float32)
    zro = jnp.zeros((_L,), jnp.float32)
    t1 = [np.float32(k * _WIDTH) for k in range(1, _BINS)]
    t2 = [np.float32((k + 1.0) / _BINS) for k in range(1, _BINS)]

    def body(i, carry):
        g1 = list(carry[:_BINS - 1])
        g2 = list(carry[_BINS - 1:])
        for t in range(u):
            ds = pl.ds((i * u + t) * _L, _L)
            m = bm[ds]
            ch = (buf[ds] * m - tmin_s) * scale_s
            for k in range(_BINS - 1):
                g1[k] = g1[k] + jnp.where(ch >= t1[k], one, zro)
                g2[k] = g2[k] + jnp.where(ch >= t2[k], one, zro)
            h = jnp.clip((ch * 6.0 - 1.0).astype(jnp.int32), 0, 5)
            hid[ds] = jnp.where(m == 0.0, _BINS, h)
        return tuple(g1) + tuple(g2)

    out = lax.fori_loop(0, nchunk // u, body, (zro,) * (2 * (_BINS - 1)))
    gg1 = [_splat_sum(out[k]) for k in range(_BINS - 1)]
    gg2 = [_splat_sum(out[_BINS - 1 + k]) for k in range(_BINS - 1)]
    n_s = jnp.full((_L,), nf, jnp.float32)
    c1 = ([n_s - gg1[0]] + [gg1[k] - gg1[k + 1] for k in range(_BINS - 2)]
          + [gg1[_BINS - 2]])
    c2 = ([n_s - gg2[0]] + [gg2[k] - gg2[k + 1] for k in range(_BINS - 2)]
          + [gg2[_BINS - 2]])
    return c1, c2


def _chan_table(c1i, c2i, tmin_s, scale_s, n, nb, is0_s):
    c1 = list(c1i)
    c2 = list(c2i)
    chb = (0.0 - tmin_s) * scale_s
    hb = jnp.clip((chb * 6.0 - 1.0).astype(jnp.int32), 0, 5)

    lv = [-_ln16(c1[b] * (1.0 / n) + 1e-4) for b in range(_BINS)]
    pres = [c2[b] > 0.0 for b in range(_BINS)]
    dmin = _fmin6([jnp.where(pres[b], lv[b], _BIG) for b in range(_BINS)])
    dmax = _fmax6([jnp.where(pres[b], lv[b], -_BIG) for b in range(_BINS)])
    drng = dmax - dmin
    ddeg = drng == 0.0
    dsafe = jnp.where(ddeg, 1.0, drng)
    ln = [jnp.where(ddeg, 0.0, (lv[b] - dmin) / dsafe) for b in range(_BINS)]
    lmax = _fmax6([jnp.where(pres[b], ln[b], -_BIG) for b in range(_BINS)])
    lmean = sum(c2[b] * ln[b] for b in range(_BINS)) * (1.0 / n)
    w_r = (lmax - lmean) * (lmax - lmean)
    rv = [ln[b] * w_r for b in range(_BINS)]

    rminp = _fmin6([jnp.where(pres[b], rv[b], _BIG) for b in range(_BINS)])
    rmaxp = _fmax6([jnp.where(pres[b], rv[b], -_BIG) for b in range(_BINS)])
    rmean = sum(c2[b] * rv[b] for b in range(_BINS)) * (1.0 / n)
    w0 = (rmaxp - rmean) * (rmaxp - rmean)
    frng = rmaxp - rminp
    fdeg = frng == 0.0
    fsafe = jnp.where(fdeg, 1.0, frng)
    t0 = [jnp.where(fdeg, 0.0, (rv[b] - rminp) / fsafe * w0)
          for b in range(_BINS)]
    sb0 = sum(jnp.where(hb == b, t0[b], 0.0) for b in range(_BINS))

    cint = [c2[b] - nb * jnp.where(hb == b, 1.0, 0.0) for b in range(_BINS)]
    presi = [cint[b] > 0.0 for b in range(_BINS)]
    zmin = jnp.minimum(
        0.0, _fmin6([jnp.where(presi[b], rv[b], _BIG) for b in range(_BINS)]))
    zmax = jnp.maximum(
        0.0, _fmax6([jnp.where(presi[b], rv[b], -_BIG) for b in range(_BINS)]))
    zmean = sum(cint[b] * rv[b] for b in range(_BINS)) * (1.0 / n)
    wz = (zmax - zmean) * (zmax - zmean)
    zrng = zmax - zmin
    zdeg = zrng == 0.0
    zsafe = jnp.where(zdeg, 1.0, zrng)
    tz = [jnp.where(zdeg, 0.0, (rv[b] - zmin) / zsafe * wz)
          for b in range(_BINS)]
    bz = jnp.where(zdeg, 0.0, (0.0 - zmin) / zsafe * wz)

    tab = [tz[b] + (t0[b] - tz[b]) * is0_s for b in range(_BINS)]
    sb = bz + (sb0 - bz) * is0_s
    return tab, sb


def _apply_pass(hid, acc, nchunk, u, tabv):
    def body(i, _):
        for t in range(u):
            ds = pl.ds((i * u + t) * _L, _L)
            acc[ds] = acc[ds] + tabv.at[hid[ds]].get(
                mode="promise_in_bounds")
        return 0

    lax.fori_loop(0, nchunk // u, body, 0)


def _sc_layer(hbm, bmh, out, buf, bm, hid, acc, wid, hdim, wdim, per, u):
    hw = hdim * wdim
    nchunk = hw // _L
    nf = float(hw)
    nbf = float(2 * hdim + 2 * wdim - 4)

    def zero_body(i, _):
        for t in range(u):
            ds = pl.ds((i * u + t) * _L, _L)
            acc[ds] = jnp.zeros((_L,), jnp.float32)
        return 0

    lax.fori_loop(0, nchunk // u, zero_body, 0)
    pltpu.sync_copy(bmh, bm)

    def chan_body(k, _):
        c = wid * per + k
        pltpu.sync_copy(hbm.at[pl.ds(c * hw, hw)], buf)
        tmin_s, tmax_s = _minmax_pass(buf, bm, nchunk, u)
        rng = tmax_s - tmin_s
        deg = rng == 0.0
        scale_s = jnp.where(deg, 0.0, 256.0 / jnp.where(deg, 1.0, rng))
        c1, c2 = _hist_pass(buf, bm, hid, nchunk, u, tmin_s, scale_s, nf)
        is0_s = jnp.where(c == 0, 1.0, 0.0).astype(jnp.float32)
        tab, sb = _chan_table(c1, c2, tmin_s, scale_s, nf, nbf, is0_s)
        lane = lax.broadcasted_iota(jnp.int32, (_L,), 0)
        tabv = sb
        for b in range(_BINS):
            tabv = jnp.where(lane == b, tab[b], tabv)
        _apply_pass(hid, acc, nchunk, u, tabv)
        return 0

    lax.fori_loop(0, per, chan_body, 0)
    pltpu.sync_copy(acc, out.at[pl.ds(wid * hw, hw)])


def _make_sc_call():
    mesh = plsc.VectorSubcoreMesh(core_axis_name="c", subcore_axis_name="s")
    out_type = [jax.ShapeDtypeStruct((_NW * h * w,), jnp.float32)
                for h, w, _, _ in _LAYERS]
    scratch = []
    for h, w, _, _ in _LAYERS:
        hw = h * w
        scratch += [pltpu.VMEM((hw,), jnp.float32),
                    pltpu.VMEM((hw,), jnp.float32),
                    pltpu.VMEM((hw,), jnp.int32),
                    pltpu.VMEM((hw,), jnp.float32)]

    @functools.partial(pl.kernel, mesh=mesh, out_type=out_type,
                       scratch_types=scratch)
    def body(l0, l1, l2, mb0, mb1, mb2, p0, p1, p2,
             buf0, bm0, hid0, acc0,
             buf1, bm1, hid1, acc1,
             buf2, bm2, hid2, acc2):
        wid = lax.axis_index("s") * 2 + lax.axis_index("c")
        hbms = (l0, l1, l2)
        masks = (mb0, mb1, mb2)
        outs = (p0, p1, p2)
        scr = ((buf0, bm0, hid0, acc0),
               (buf1, bm1, hid1, acc1),
               (buf2, bm2, hid2, acc2))
        for li, (hdim, wdim, per, u) in enumerate(_LAYERS):
            buf, bmm, hid, acc = scr[li]
            _sc_layer(hbms[li], masks[li], outs[li], buf, bmm, hid, acc,
                      wid, hdim, wdim, per, u)

    return body


_SC_CALL = _make_sc_call()


def _tail_body(p_ref, a_ref, at_ref, col_ref):
    p = jnp.sum(p_ref[...], axis=0)
    pmin = jnp.min(p)
    pmax = jnp.max(p)
    prng = pmax - pmin
    pdeg = prng == 0.0
    psafe = jnp.where(pdeg, 1.0, prng)
    pn = jnp.where(pdeg, 0.0, (p - pmin) / psafe)
    pt = jnp.where(pn < 0.2, 0.0, pn)
    tmp = jnp.dot(a_ref[...], pt, preferred_element_type=jnp.float32)
    r = jnp.dot(tmp, at_ref[...], preferred_element_type=jnp.float32)
    rmin = jnp.min(r)
    rmax = jnp.max(r)
    rrng = rmax - rmin
    rdeg = rrng == 0.0
    rsafe = jnp.where(rdeg, 1.0, rrng)
    col_ref[...] = jnp.where(rdeg, 0.0, (r - rmin) / rsafe * 256.0)


def _tail_col(p, amat, atmat):
    nw, h, w = p.shape
    return pl.pallas_call(
        _tail_body,
        grid=(1,),
        in_specs=[
            pl.BlockSpec((nw, h, w), lambda i: (0, 0, 0)),
            pl.BlockSpec((_OUT, h), lambda i: (0, 0)),
            pl.BlockSpec((h, _OUT), lambda i: (0, 0)),
        ],
        out_specs=pl.BlockSpec((_OUT, _OUT), lambda i: (0, 0)),
        out_shape=jax.ShapeDtypeStruct((_OUT, _OUT), jnp.float32),
    )(p, amat, atmat)


_A112 = _resize_matrix(112)
_A56 = _resize_matrix(56)
_A28 = _resize_matrix(28)


def _border_mask(h, w):
    m = np.ones((h, w), np.float32)
    m[0, :] = 0.0
    m[-1, :] = 0.0
    m[:, 0] = 0.0
    m[:, -1] = 0.0
    return m.reshape(-1)


_BMASKS = tuple(_border_mask(h, w) for h, w, _, _ in _LAYERS)


def kernel(layer0, layer1, layer2):
    flats = [x[0].reshape(x.shape[1] * x.shape[2] * x.shape[3])
             for x in (layer0, layer1, layer2)]
    parts = _SC_CALL(*flats, *(jnp.asarray(m) for m in _BMASKS))
    cols = []
    for part, (h, w, _, _), a in zip(parts, _LAYERS, (_A112, _A56, _A28)):
        cols.append(_tail_col(
            part.reshape(_NW, h, w),
            jnp.asarray(a), jnp.asarray(np.ascontiguousarray(a.T))))
    groups = jnp.stack(cols, axis=-1)
    return groups.sum(axis=-1), groups


# speedup vs baseline: 1.8089x; 1.5106x over previous
import functools

import numpy as np
import jax
import jax.numpy as jnp
from jax import lax
from jax.experimental import pallas as pl
from jax.experimental.pallas import tpu as pltpu
from jax.experimental.pallas import tpu_sc as plsc

_BINS = 6
_WIDTH = np.float32(256.0 / _BINS)
_OUT = 240
_BIG = np.float32(3.0e38)
_NW = 32
_L = 16

_LAYERS = ((112, 112, 1, 8), (56, 56, 2, 7), (28, 28, 4, 7))
_SC_CNT = tuple(_NW * per for _, _, per, _ in _LAYERS)

_LOG2C = (-0.02482598, 0.26686277, -1.2342799, 3.21886981,
          -5.26415552, 6.06585886, -3.02832497)
_LN2 = 0.6931471805599453


def _resize_matrix(src):
    x = (np.arange(_OUT, dtype=np.float64) + 0.5) * (src / _OUT) - 0.5
    lo = np.floor(x).astype(np.int64)
    frac = x - lo
    a = np.zeros((_OUT, src), np.float64)
    for i in range(_OUT):
        for tap, wt in ((lo[i], 1.0 - frac[i]), (lo[i] + 1, frac[i])):
            a[i, min(max(int(tap), 0), src - 1)] += wt
    return a.astype(np.float32)


def _fmin6(vals):
    return functools.reduce(jnp.minimum, vals)


def _fmax6(vals):
    return functools.reduce(jnp.maximum, vals)


def _ln16(x):
    bits = lax.bitcast_convert_type(x, jnp.int32)
    e = ((bits >> 23) & 0xFF).astype(jnp.float32) - 127.0
    m = lax.bitcast_convert_type((bits & 0x007FFFFF) | 0x3F800000,
                                 jnp.float32)
    p = jnp.full((_L,), _LOG2C[0], jnp.float32)
    for c in _LOG2C[1:]:
        p = p * m + c
    return (e + p) * _LN2


def _lane_shuffle(v, sh):
    idx = lax.broadcasted_iota(jnp.int32, (_L,), 0) ^ sh
    return v.at[idx].get(mode="promise_in_bounds")


def _splat_min(v):
    for sh in (8, 4, 2, 1):
        v = jnp.minimum(v, _lane_shuffle(v, sh))
    return v


def _splat_max(v):
    for sh in (8, 4, 2, 1):
        v = jnp.maximum(v, _lane_shuffle(v, sh))
    return v


def _minmax_pass(buf, bm, nchunk, u):
    def body(i, carry):
        mn, mx = carry
        for t in range(u):
            ds = pl.ds((i * u + t) * _L, _L)
            v0 = buf[ds] * bm[ds]
            mn = jnp.minimum(mn, v0)
            mx = jnp.maximum(mx, v0)
        return mn, mx

    init = jnp.full((_L,), _BIG, jnp.float32)
    mn, mx = lax.fori_loop(0, nchunk // u, body, (init, -init))
    return _splat_min(mn), _splat_max(mx)


def _splat_sum(v):
    for sh in (8, 4, 2, 1):
        v = v + _lane_shuffle(v, sh)
    return v


def _hist_pass(buf, bm, hid, nchunk, u, tmin_s, scale_s, nf):
    one = jnp.full((_L,), 1.0, jnp.float32)
    zro = jnp.zeros((_L,), jnp.float32)
    t1 = [np.float32(k * _WIDTH) for k in range(1, _BINS)]
    t2 = [np.float32((k + 1.0) / _BINS) for k in range(1, _BINS)]

    def body(i, carry):
        g1 = list(carry[:_BINS - 1])
        g2 = list(carry[_BINS - 1:])
        for t in range(u):
            ds = pl.ds((i * u + t) * _L, _L)
            m = bm[ds]
            ch = (buf[ds] * m - tmin_s) * scale_s
            for k in range(_BINS - 1):
                g1[k] = g1[k] + jnp.where(ch >= t1[k], one, zro)
                g2[k] = g2[k] + jnp.where(ch >= t2[k], one, zro)
            h = jnp.clip((ch * 6.0 - 1.0).astype(jnp.int32), 0, 5)
            hid[ds] = jnp.where(m == 0.0, _BINS, h)
        return tuple(g1) + tuple(g2)

    out = lax.fori_loop(0, nchunk // u, body, (zro,) * (2 * (_BINS - 1)))
    gg1 = [_splat_sum(out[k]) for k in range(_BINS - 1)]
    gg2 = [_splat_sum(out[_BINS - 1 + k]) for k in range(_BINS - 1)]
    n_s = jnp.full((_L,), nf, jnp.float32)
    c1 = ([n_s - gg1[0]] + [gg1[k] - gg1[k + 1] for k in range(_BINS - 2)]
          + [gg1[_BINS - 2]])
    c2 = ([n_s - gg2[0]] + [gg2[k] - gg2[k + 1] for k in range(_BINS - 2)]
          + [gg2[_BINS - 2]])
    return c1, c2


def _chan_table(c1i, c2i, tmin_s, scale_s, n, nb, is0_s):
    c1 = list(c1i)
    c2 = list(c2i)
    chb = (0.0 - tmin_s) * scale_s
    hb = jnp.clip((chb * 6.0 - 1.0).astype(jnp.int32), 0, 5)

    lv = [-_ln16(c1[b] * (1.0 / n) + 1e-4) for b in range(_BINS)]
    pres = [c2[b] > 0.0 for b in range(_BINS)]
    dmin = _fmin6([jnp.where(pres[b], lv[b], _BIG) for b in range(_BINS)])
    dmax = _fmax6([jnp.where(pres[b], lv[b], -_BIG) for b in range(_BINS)])
    drng = dmax - dmin
    ddeg = drng == 0.0
    dsafe = jnp.where(ddeg, 1.0, drng)
    ln = [jnp.where(ddeg, 0.0, (lv[b] - dmin) / dsafe) for b in range(_BINS)]
    lmax = _fmax6([jnp.where(pres[b], ln[b], -_BIG) for b in range(_BINS)])
    lmean = sum(c2[b] * ln[b] for b in range(_BINS)) * (1.0 / n)
    w_r = (lmax - lmean) * (lmax - lmean)
    rv = [ln[b] * w_r for b in range(_BINS)]

    rminp = _fmin6([jnp.where(pres[b], rv[b], _BIG) for b in range(_BINS)])
    rmaxp = _fmax6([jnp.where(pres[b], rv[b], -_BIG) for b in range(_BINS)])
    rmean = sum(c2[b] * rv[b] for b in range(_BINS)) * (1.0 / n)
    w0 = (rmaxp - rmean) * (rmaxp - rmean)
    frng = rmaxp - rminp
    fdeg = frng == 0.0
    fsafe = jnp.where(fdeg, 1.0, frng)
    t0 = [jnp.where(fdeg, 0.0, (rv[b] - rminp) / fsafe * w0)
          for b in range(_BINS)]
    sb0 = sum(jnp.where(hb == b, t0[b], 0.0) for b in range(_BINS))

    cint = [c2[b] - nb * jnp.where(hb == b, 1.0, 0.0) for b in range(_BINS)]
    presi = [cint[b] > 0.0 for b in range(_BINS)]
    zmin = jnp.minimum(
        0.0, _fmin6([jnp.where(presi[b], rv[b], _BIG) for b in range(_BINS)]))
    zmax = jnp.maximum(
        0.0, _fmax6([jnp.where(presi[b], rv[b], -_BIG) for b in range(_BINS)]))
    zmean = sum(cint[b] * rv[b] for b in range(_BINS)) * (1.0 / n)
    wz = (zmax - zmean) * (zmax - zmean)
    zrng = zmax - zmin
    zdeg = zrng == 0.0
    zsafe = jnp.where(zdeg, 1.0, zrng)
    tz = [jnp.where(zdeg, 0.0, (rv[b] - zmin) / zsafe * wz)
          for b in range(_BINS)]
    bz = jnp.where(zdeg, 0.0, (0.0 - zmin) / zsafe * wz)

    tab = [tz[b] + (t0[b] - tz[b]) * is0_s for b in range(_BINS)]
    sb = bz + (sb0 - bz) * is0_s
    return tab, sb


def _apply_pass(hid, acc, nchunk, u, tabv):
    def body(i, _):
        for t in range(u):
            ds = pl.ds((i * u + t) * _L, _L)
            acc[ds] = acc[ds] + tabv.at[hid[ds]].get(
                mode="promise_in_bounds")
        return 0

    lax.fori_loop(0, nchunk // u, body, 0)


def _sc_layer(hbm, bmh, out, buf, bm, hid, acc, wid, hdim, wdim, per, u):
    hw = hdim * wdim
    nchunk = hw // _L
    nf = float(hw)
    nbf = float(2 * hdim + 2 * wdim - 4)

    def zero_body(i, _):
        for t in range(u):
            ds = pl.ds((i * u + t) * _L, _L)
            acc[ds] = jnp.zeros((_L,), jnp.float32)
        return 0

    lax.fori_loop(0, nchunk // u, zero_body, 0)
    pltpu.sync_copy(bmh, bm)

    def chan_body(k, _):
        c = wid * per + k
        pltpu.sync_copy(hbm.at[pl.ds(c * hw, hw)], buf)
        tmin_s, tmax_s = _minmax_pass(buf, bm, nchunk, u)
        rng = tmax_s - tmin_s
        deg = rng == 0.0
        scale_s = jnp.where(deg, 0.0, 256.0 / jnp.where(deg, 1.0, rng))
        c1, c2 = _hist_pass(buf, bm, hid, nchunk, u, tmin_s, scale_s, nf)
        is0_s = jnp.where(c == 0, 1.0, 0.0).astype(jnp.float32)
        tab, sb = _chan_table(c1, c2, tmin_s, scale_s, nf, nbf, is0_s)
        lane = lax.broadcasted_iota(jnp.int32, (_L,), 0)
        tabv = sb
        for b in range(_BINS):
            tabv = jnp.where(lane == b, tab[b], tabv)
        _apply_pass(hid, acc, nchunk, u, tabv)
        return 0

    lax.fori_loop(0, per, chan_body, 0)
    pltpu.sync_copy(acc, out.at[pl.ds(wid * hw, hw)])


def _make_sc_call():
    mesh = plsc.VectorSubcoreMesh(core_axis_name="c", subcore_axis_name="s")
    out_type = [jax.ShapeDtypeStruct((_NW * h * w,), jnp.float32)
                for h, w, _, _ in _LAYERS]
    scratch = []
    for h, w, _, _ in _LAYERS:
        hw = h * w
        scratch += [pltpu.VMEM((hw,), jnp.float32),
                    pltpu.VMEM((hw,), jnp.float32),
                    pltpu.VMEM((hw,), jnp.int32),
                    pltpu.VMEM((hw,), jnp.float32)]

    @functools.partial(pl.kernel, mesh=mesh, out_type=out_type,
                       scratch_types=scratch)
    def body(l0, l1, l2, mb0, mb1, mb2, p0, p1, p2,
             buf0, bm0, hid0, acc0,
             buf1, bm1, hid1, acc1,
             buf2, bm2, hid2, acc2):
        wid = lax.axis_index("s") * 2 + lax.axis_index("c")
        hbms = (l0, l1, l2)
        masks = (mb0, mb1, mb2)
        outs = (p0, p1, p2)
        scr = ((buf0, bm0, hid0, acc0),
               (buf1, bm1, hid1, acc1),
               (buf2, bm2, hid2, acc2))
        for li, (hdim, wdim, per, u) in enumerate(_LAYERS):
            buf, bmm, hid, acc = scr[li]
            _sc_layer(hbms[li], masks[li], outs[li], buf, bmm, hid, acc,
                      wid, hdim, wdim, per, u)

    return body


_SC_CALL = _make_sc_call()


def _tcpart_body(x_ref, p_ref):
    t = x_ref[...]
    cdim, h, w = t.shape
    rows = lax.broadcasted_iota(jnp.int32, (h, w), 0)
    cols = lax.broadcasted_iota(jnp.int32, (h, w), 1)
    border = ((rows == 0) | (rows == h - 1) | (cols == 0) | (cols == w - 1))
    border = border[None]
    t = jnp.where(border, jnp.float32(0.0), t)

    def rsum(x):
        return jnp.sum(x, axis=(1, 2), keepdims=True)

    tmin = jnp.min(t, axis=(1, 2), keepdims=True)
    tmax = jnp.max(t, axis=(1, 2), keepdims=True)
    rng = tmax - tmin
    deg = rng == 0.0
    safe = jnp.where(deg, jnp.float32(1.0), rng)
    ch = jnp.where(deg, 0.0, (t - tmin) / safe * 256.0)
    chb = jnp.where(deg, 0.0, (0.0 - tmin) / safe * 256.0)

    bin1 = jnp.clip(jnp.floor(ch / _WIDTH), 0.0, 5.0)
    hidx = jnp.clip(jnp.floor(ch * 6.0 - 1.0), 0.0, 5.0)
    hb = jnp.clip(jnp.floor(chb * 6.0 - 1.0), 0.0, 5.0)

    n = jnp.float32(h * w)
    nb = jnp.float32(2 * h + 2 * w - 4)

    c1 = [rsum(jnp.where(bin1 == b, 1.0, 0.0)) for b in range(_BINS)]
    c2 = [rsum(jnp.where(hidx == b, 1.0, 0.0)) for b in range(_BINS)]

    lv = [-jnp.log(c1[b] / n + 1e-4) for b in range(_BINS)]
    pres = [c2[b] > 0.0 for b in range(_BINS)]
    dmin = _fmin6([jnp.where(pres[b], lv[b], _BIG) for b in range(_BINS)])
    dmax = _fmax6([jnp.where(pres[b], lv[b], -_BIG) for b in range(_BINS)])
    drng = dmax - dmin
    ddeg = drng == 0.0
    dsafe = jnp.where(ddeg, 1.0, drng)
    ln = [jnp.where(ddeg, 0.0, (lv[b] - dmin) / dsafe) for b in range(_BINS)]
    lmax = _fmax6([jnp.where(pres[b], ln[b], -_BIG) for b in range(_BINS)])
    lmean = sum(c2[b] * ln[b] for b in range(_BINS)) / n
    w_r = (lmax - lmean) ** 2
    rv = [ln[b] * w_r for b in range(_BINS)]

    cint = [c2[b] - nb * jnp.where(hb == b, 1.0, 0.0) for b in range(_BINS)]
    presi = [cint[b] > 0.0 for b in range(_BINS)]
    zmin = jnp.minimum(
        0.0, _fmin6([jnp.where(presi[b], rv[b], _BIG) for b in range(_BINS)]))
    zmax = jnp.maximum(
        0.0, _fmax6([jnp.where(presi[b], rv[b], -_BIG) for b in range(_BINS)]))
    zmean = sum(cint[b] * rv[b] for b in range(_BINS)) / n
    wz = (zmax - zmean) ** 2
    zrng = zmax - zmin
    zdeg = zrng == 0.0
    zsafe = jnp.where(zdeg, 1.0, zrng)
    tz = [jnp.where(zdeg, 0.0, (rv[b] - zmin) / zsafe * wz)
          for b in range(_BINS)]
    bz = jnp.where(zdeg, 0.0, (0.0 - zmin) / zsafe * wz)

    g = tz[5]
    for b in (4, 3, 2, 1, 0):
        g = jnp.where(hidx == b, tz[b], g)
    contrib = jnp.where(border, bz, g)
    p_ref[...] = jnp.sum(contrib, axis=0)


def _tcpart_call(x):
    cdim, h, w = x.shape
    return pl.pallas_call(
        _tcpart_body,
        grid=(1,),
        in_specs=[pl.BlockSpec((cdim, h, w), lambda i: (0, 0, 0))],
        out_specs=pl.BlockSpec((h, w), lambda i: (0, 0)),
        out_shape=jax.ShapeDtypeStruct((h, w), jnp.float32),
    )(x)


def _tail_body(p_ref, tcp_ref, a_ref, at_ref, col_ref):
    p = jnp.sum(p_ref[...], axis=0) + tcp_ref[...]
    pmin = jnp.min(p)
    pmax = jnp.max(p)
    prng = pmax - pmin
    pdeg = prng == 0.0
    psafe = jnp.where(pdeg, 1.0, prng)
    pn = jnp.where(pdeg, 0.0, (p - pmin) / psafe)
    pt = jnp.where(pn < 0.2, 0.0, pn)
    tmp = jnp.dot(a_ref[...], pt, preferred_element_type=jnp.float32)
    r = jnp.dot(tmp, at_ref[...], preferred_element_type=jnp.float32)
    rmin = jnp.min(r)
    rmax = jnp.max(r)
    rrng = rmax - rmin
    rdeg = rrng == 0.0
    rsafe = jnp.where(rdeg, 1.0, rrng)
    col_ref[...] = jnp.where(rdeg, 0.0, (r - rmin) / rsafe * 256.0)


def _tail_col(p, tcp, amat, atmat):
    nw, h, w = p.shape
    return pl.pallas_call(
        _tail_body,
        grid=(1,),
        in_specs=[
            pl.BlockSpec((nw, h, w), lambda i: (0, 0, 0)),
            pl.BlockSpec((h, w), lambda i: (0, 0)),
            pl.BlockSpec((_OUT, h), lambda i: (0, 0)),
            pl.BlockSpec((h, _OUT), lambda i: (0, 0)),
        ],
        out_specs=pl.BlockSpec((_OUT, _OUT), lambda i: (0, 0)),
        out_shape=jax.ShapeDtypeStruct((_OUT, _OUT), jnp.float32),
    )(p, tcp, amat, atmat)


_A112 = _resize_matrix(112)
_A56 = _resize_matrix(56)
_A28 = _resize_matrix(28)


def _border_mask(h, w):
    m = np.ones((h, w), np.float32)
    m[0, :] = 0.0
    m[-1, :] = 0.0
    m[:, 0] = 0.0
    m[:, -1] = 0.0
    return m.reshape(-1)


_BMASKS = tuple(_border_mask(h, w) for h, w, _, _ in _LAYERS)


def kernel(layer0, layer1, layer2):
    xs = (layer0[0], layer1[0], layer2[0])
    flats = [x.reshape(-1) for x in xs]
    parts = _SC_CALL(*flats, *(jnp.asarray(m) for m in _BMASKS))
    tcps = [_tcpart_call(x[cnt:]) for x, cnt in zip(xs, _SC_CNT)]
    cols = []
    for part, tcp, (h, w, _, _), a in zip(parts, tcps, _LAYERS,
                                          (_A112, _A56, _A28)):
        cols.append(_tail_col(
            part.reshape(_NW, h, w), tcp,
            jnp.asarray(a), jnp.asarray(np.ascontiguousarray(a.T))))
    groups = jnp.stack(cols, axis=-1)
    return groups.sum(axis=-1), groups
